# baseline (speedup 1.0000x reference)
import functools

import jax
import jax.numpy as jnp
from jax import lax
from jax.experimental import pallas as pl
from jax.experimental.pallas import tpu as pltpu
from jax.experimental.pallas import tpu_sc as plsc

E = 320000
D = 128
R0, R1, R2 = 5, 6, 2
NLUT = R0 * R1 * R2

NC, NS = 2, 16
NW = NC * NS
PER_W = E // NW
CHUNK = 80
NCHUNK = PER_W // CHUNK


def _lut_body(w0_ref, w1_ref, w2_ref, out_ref):
    c = jax.lax.broadcasted_iota(jnp.int32, (NLUT, 1), 0)
    acc = None
    for rows, w_ref in ((c // (R1 * R2), w0_ref),
                        ((c // R2) % R1, w1_ref),
                        (c % R2, w2_ref)):
        n = w_ref.shape[0]
        oh = (rows == jax.lax.broadcasted_iota(jnp.int32, (1, n), 1)
              ).astype(jnp.float32)
        part = jax.lax.dot_general(oh, w_ref[...], (((1,), (0,)), ((), ())),
                                   preferred_element_type=jnp.float32)
        acc = part if acc is None else acc + part
    out_ref[...] = acc


def _build_lut(W0, W1, W2):
    return pl.pallas_call(
        _lut_body,
        out_shape=jax.ShapeDtypeStruct((NLUT, D), jnp.float32),
    )(W0, W1, W2)


def _sc_body(a0_hbm, a1_hbm, a2_hbm, lut_hbm, out_hbm,
             a0_v, a1_v, a2_v, code_v, rows_v, sem):
    wid = lax.axis_index("s") * NC + lax.axis_index("c")
    my_base = wid * PER_W

    def chunk_step(ci, _):
        base = my_base + ci * CHUNK
        pltpu.sync_copy(a0_hbm.at[pl.ds(base, CHUNK)], a0_v)
        pltpu.sync_copy(a1_hbm.at[pl.ds(base, CHUNK)], a1_v)
        pltpu.sync_copy(a2_hbm.at[pl.ds(base, CHUNK)], a2_v)
        for g in range(CHUNK // 16):
            s = pl.ds(g * 16, 16)
            code_v[s] = (a0_v[s] * (R1 * R2) + a1_v[s] * R2 + a2_v[s])
        pltpu.async_copy(lut_hbm.at[code_v], rows_v, sem).wait()
        pltpu.sync_copy(rows_v, out_hbm.at[pl.ds(base, CHUNK)])
        return ()

    lax.fori_loop(0, NCHUNK, chunk_step, ())


def kernel(edge_attr, W0, W1, W2):
    lut = _build_lut(W0, W1, W2)
    ea_t = edge_attr.T
    sc = functools.partial(
        pl.kernel,
        out_type=jax.ShapeDtypeStruct((E, D), jnp.float32),
        mesh=plsc.VectorSubcoreMesh(core_axis_name="c", subcore_axis_name="s",
                                    num_cores=NC, num_subcores=NS),
        scratch_types=[
            pltpu.VMEM((CHUNK,), jnp.int32),
            pltpu.VMEM((CHUNK,), jnp.int32),
            pltpu.VMEM((CHUNK,), jnp.int32),
            pltpu.VMEM((CHUNK,), jnp.int32),
            pltpu.VMEM((CHUNK, D), jnp.float32),
            pltpu.SemaphoreType.DMA,
        ],
    )(_sc_body)
    return sc(ea_t[0], ea_t[1], ea_t[2], lut)


# speedup vs baseline: 1.0118x; 1.0118x over previous
import functools

import jax
import jax.numpy as jnp
from jax import lax
from jax.experimental import pallas as pl
from jax.experimental.pallas import tpu as pltpu
from jax.experimental.pallas import tpu_sc as plsc

E = 320000
D = 128
R0, R1, R2 = 5, 6, 2
NLUT = R0 * R1 * R2

NC, NS = 2, 16
NW = NC * NS
PER_W = E // NW
CHUNK = 80
NCHUNK = PER_W // CHUNK
NBUF = 5


def _lut_body(w0_ref, w1_ref, w2_ref, out_ref):
    c = jax.lax.broadcasted_iota(jnp.int32, (NLUT, 1), 0)
    acc = None
    for rows, w_ref in ((c // (R1 * R2), w0_ref),
                        ((c // R2) % R1, w1_ref),
                        (c % R2, w2_ref)):
        n = w_ref.shape[0]
        oh = (rows == jax.lax.broadcasted_iota(jnp.int32, (1, n), 1)
              ).astype(jnp.float32)
        part = jax.lax.dot_general(oh, w_ref[...], (((1,), (0,)), ((), ())),
                                   preferred_element_type=jnp.float32)
        acc = part if acc is None else acc + part
    out_ref[...] = acc


def _build_lut(W0, W1, W2):
    return pl.pallas_call(
        _lut_body,
        out_shape=jax.ShapeDtypeStruct((NLUT, D), jnp.float32),
    )(W0, W1, W2)


def _sc_body(a0_hbm, a1_hbm, a2_hbm, lut_hbm, out_hbm,
             a0_v, a1_v, a2_v, code_v, rows_v, gsems, ssems):
    wid = lax.axis_index("s") * NC + lax.axis_index("c")
    my_base = wid * PER_W

    pltpu.sync_copy(a0_hbm.at[pl.ds(my_base, PER_W)], a0_v)
    pltpu.sync_copy(a1_hbm.at[pl.ds(my_base, PER_W)], a1_v)
    pltpu.sync_copy(a2_hbm.at[pl.ds(my_base, PER_W)], a2_v)

    def code_step(i, _):
        for j in range(5):
            s = pl.ds(i * 80 + j * 16, 16)
            code_v[s] = a0_v[s] * (R1 * R2) + a1_v[s] * R2 + a2_v[s]
        return ()
    lax.fori_loop(0, PER_W // 80, code_step, ())

    def gather(n, b):
        return pltpu.make_async_copy(
            lut_hbm.at[code_v.at[pl.ds(n * CHUNK, CHUNK)]], rows_v[b],
            gsems[b])

    def scatter(n, b):
        return pltpu.make_async_copy(
            rows_v[b], out_hbm.at[pl.ds(my_base + n * CHUNK, CHUNK)],
            ssems[b])

    for b in range(NBUF):
        gather(b, b).start()

    def ring_step(i, _):
        for b in range(NBUF):
            n = i * NBUF + b
            gather(n, b).wait()
            sc = scatter(n, b)
            sc.start()
            sc.wait()
            gather(n + NBUF, b).start()
        return ()
    lax.fori_loop(0, NCHUNK // NBUF - 1, ring_step, ())

    for b in range(NBUF):
        n = NCHUNK - NBUF + b
        gather(n, b).wait()
        sc = scatter(n, b)
        sc.start()
        sc.wait()


def kernel(edge_attr, W0, W1, W2):
    lut = _build_lut(W0, W1, W2)
    ea_t = edge_attr.T
    sc = functools.partial(
        pl.kernel,
        out_type=jax.ShapeDtypeStruct((E, D), jnp.float32),
        mesh=plsc.VectorSubcoreMesh(core_axis_name="c", subcore_axis_name="s",
                                    num_cores=NC, num_subcores=NS),
        scratch_types=[
            pltpu.VMEM((PER_W,), jnp.int32),
            pltpu.VMEM((PER_W,), jnp.int32),
            pltpu.VMEM((PER_W,), jnp.int32),
            pltpu.VMEM((PER_W,), jnp.int32),
            [pltpu.VMEM((CHUNK, D), jnp.float32) for _ in range(NBUF)],
            [pltpu.SemaphoreType.DMA for _ in range(NBUF)],
            [pltpu.SemaphoreType.DMA for _ in range(NBUF)],
        ],
    )(_sc_body)
    return sc(ea_t[0], ea_t[1], ea_t[2], lut)


# speedup vs baseline: 1.0119x; 1.0000x over previous
import functools

import jax
import jax.numpy as jnp
from jax import lax
from jax.experimental import pallas as pl
from jax.experimental.pallas import tpu as pltpu
from jax.experimental.pallas import tpu_sc as plsc

E = 320000
D = 128
R0, R1, R2 = 5, 6, 2
NLUT = R0 * R1 * R2

NC, NS = 2, 16
NW = NC * NS
PER_W = E // NW
CHUNK = 80
NCHUNK = PER_W // CHUNK
NBUF = 5
P = 2


def _lut_body(w0_ref, w1_ref, w2_ref, out_ref):
    c = jax.lax.broadcasted_iota(jnp.int32, (NLUT, 1), 0)
    acc = None
    for rows, w_ref in ((c // (R1 * R2), w0_ref),
                        ((c // R2) % R1, w1_ref),
                        (c % R2, w2_ref)):
        n = w_ref.shape[0]
        oh = (rows == jax.lax.broadcasted_iota(jnp.int32, (1, n), 1)
              ).astype(jnp.float32)
        part = jax.lax.dot_general(oh, w_ref[...], (((1,), (0,)), ((), ())),
                                   preferred_element_type=jnp.float32)
        acc = part if acc is None else acc + part
    out_ref[...] = acc


def _build_lut(W0, W1, W2):
    return pl.pallas_call(
        _lut_body,
        out_shape=jax.ShapeDtypeStruct((NLUT, D), jnp.float32),
    )(W0, W1, W2)


def _sc_body(a0_hbm, a1_hbm, a2_hbm, lut_hbm, out_hbm,
             a0_v, a1_v, a2_v, code_v, rows_v, gsems, ssems):
    wid = lax.axis_index("s") * NC + lax.axis_index("c")
    my_base = wid * PER_W

    pltpu.sync_copy(a0_hbm.at[pl.ds(my_base, PER_W)], a0_v)
    pltpu.sync_copy(a1_hbm.at[pl.ds(my_base, PER_W)], a1_v)
    pltpu.sync_copy(a2_hbm.at[pl.ds(my_base, PER_W)], a2_v)

    def code_step(i, _):
        for j in range(5):
            s = pl.ds(i * 80 + j * 16, 16)
            code_v[s] = a0_v[s] * (R1 * R2) + a1_v[s] * R2 + a2_v[s]
        return ()
    lax.fori_loop(0, PER_W // 80, code_step, ())

    def gather(n, b):
        return pltpu.make_async_copy(
            lut_hbm.at[code_v.at[pl.ds(n * CHUNK, CHUNK)]], rows_v[b],
            gsems[b])

    def scatter(n, b):
        return pltpu.make_async_copy(
            rows_v[b], out_hbm.at[pl.ds(my_base + n * CHUNK, CHUNK)],
            ssems[b])

    def step(n, b):
        gather(n, b).wait()
        scatter(n, b).start()
        m = n + P
        mb = (b + P) % NBUF
        if isinstance(n, int):
            if m - NBUF >= 0:
                scatter(m - NBUF, mb).wait()
            if m < NCHUNK:
                gather(m, mb).start()
        else:
            scatter(m - NBUF, mb).wait()
            gather(m, mb).start()

    for n in range(P):
        gather(n, n % NBUF).start()
    for b in range(NBUF):
        step(b, b)

    def ring_step(i, _):
        for b in range(NBUF):
            step(i * NBUF + b, b)
        return ()
    lax.fori_loop(1, NCHUNK // NBUF - 1, ring_step, ())

    for b in range(NBUF):
        step(NCHUNK - NBUF + b, b)
    for n in range(NCHUNK - NBUF + P, NCHUNK):
        scatter(n, n % NBUF).wait()


def kernel(edge_attr, W0, W1, W2):
    lut = _build_lut(W0, W1, W2)
    ea_t = edge_attr.T
    sc = functools.partial(
        pl.kernel,
        out_type=jax.ShapeDtypeStruct((E, D), jnp.float32),
        mesh=plsc.VectorSubcoreMesh(core_axis_name="c", subcore_axis_name="s",
                                    num_cores=NC, num_subcores=NS),
        scratch_types=[
            pltpu.VMEM((PER_W,), jnp.int32),
            pltpu.VMEM((PER_W,), jnp.int32),
            pltpu.VMEM((PER_W,), jnp.int32),
            pltpu.VMEM((PER_W,), jnp.int32),
            [pltpu.VMEM((CHUNK, D), jnp.float32) for _ in range(NBUF)],
            [pltpu.SemaphoreType.DMA for _ in range(NBUF)],
            [pltpu.SemaphoreType.DMA for _ in range(NBUF)],
        ],
    )(_sc_body)
    return sc(ea_t[0], ea_t[1], ea_t[2], lut)
